# baseline (speedup 1.0000x reference)
import jax
import jax.numpy as jnp
from jax.experimental import pallas as pl
from jax.experimental.pallas import tpu as pltpu

_NCHUNK = 8
_CH = 64 // _NCHUNK


def _aq_kernel(w_ref, x_ref, n_ref, o_ref, xb, nb, ob, xs, ns, os_):
    cw28 = w_ref[0]
    for i in range(1, 29):
        cw28 = cw28 + w_ref[i]
    cw29 = cw28 + w_ref[29]
    d = (cw29 - cw28) * 0.5
    a0 = -(cw29 + cw28) * 0.5
    a59 = (cw29 + cw28) * 0.5
    rt = cw29 + d

    core = pl.program_id(0)
    b0 = core * 2

    def in_copy(k, slot):
        row = k * _CH
        cx = pltpu.make_async_copy(
            x_ref.at[pl.ds(b0, 2), pl.ds(row, _CH), :, :], xb.at[slot],
            xs.at[slot])
        cn = pltpu.make_async_copy(
            n_ref.at[pl.ds(b0, 2), pl.ds(row, _CH), :, :], nb.at[slot],
            ns.at[slot])
        cx.start()
        cn.start()

    in_copy(0, 0)
    for k in range(_NCHUNK):
        slot = k % 2
        if k + 1 < _NCHUNK:
            in_copy(k + 1, 1 - slot)
        pltpu.make_async_copy(xb.at[slot], xb.at[slot], xs.at[slot]).wait()
        pltpu.make_async_copy(nb.at[slot], nb.at[slot], ns.at[slot]).wait()
        if k >= 2:
            pltpu.make_async_copy(ob.at[slot], ob.at[slot], os_.at[slot]).wait()
        x = xb[slot]
        nz = nb[slot]
        c_lo = x > a0
        v0 = jnp.where(c_lo & (x <= rt), d, 0.0)
        v1 = jnp.where(c_lo & (x <= a59), d, 0.0) + jnp.where(
            (x > rt) & (x <= a0), d, 0.0)
        ob[slot] = x + v1 - (v0 + v1) * nz
        pltpu.make_async_copy(
            ob.at[slot], o_ref.at[pl.ds(b0, 2), pl.ds(k * _CH, _CH), :, :],
            os_.at[slot]).start()
    for slot in range(2):
        pltpu.make_async_copy(ob.at[slot], ob.at[slot], os_.at[slot]).wait()


def kernel(x, noise, w):
    xt = jnp.transpose(x, (0, 2, 3, 1))
    nt = jnp.transpose(noise, (0, 2, 3, 1))
    b, h, wd, c = xt.shape
    chunk = (2, _CH, wd, c)
    out = pl.pallas_call(
        _aq_kernel,
        grid=(b // 2,),
        in_specs=[
            pl.BlockSpec(memory_space=pltpu.SMEM),
            pl.BlockSpec(memory_space=pl.ANY),
            pl.BlockSpec(memory_space=pl.ANY),
        ],
        out_specs=pl.BlockSpec(memory_space=pl.ANY),
        out_shape=jax.ShapeDtypeStruct(xt.shape, x.dtype),
        scratch_shapes=[
            pltpu.VMEM((2,) + chunk, x.dtype),
            pltpu.VMEM((2,) + chunk, x.dtype),
            pltpu.VMEM((2,) + chunk, x.dtype),
            pltpu.SemaphoreType.DMA((2,)),
            pltpu.SemaphoreType.DMA((2,)),
            pltpu.SemaphoreType.DMA((2,)),
        ],
        compiler_params=pltpu.CompilerParams(
            dimension_semantics=("parallel",)),
    )(w, xt, nt)
    return jnp.transpose(out, (0, 3, 1, 2))


# speedup vs baseline: 1.1270x; 1.1270x over previous
import jax
import jax.numpy as jnp
from jax.experimental import pallas as pl
from jax.experimental.pallas import tpu as pltpu

_NCHUNK = 4
_CH = 64 // _NCHUNK


def _aq_kernel(w_ref, x_ref, n_ref, o_ref, xb, nb, ob, xs, ns, os_):
    cw28 = w_ref[0]
    for i in range(1, 29):
        cw28 = cw28 + w_ref[i]
    cw29 = cw28 + w_ref[29]
    d = (cw29 - cw28) * 0.5
    a0 = -(cw29 + cw28) * 0.5
    a59 = (cw29 + cw28) * 0.5
    rt = cw29 + d

    core = pl.program_id(0)
    b0 = core * 2

    def in_copy(k, slot):
        row = k * _CH
        cx = pltpu.make_async_copy(
            x_ref.at[pl.ds(b0, 2), pl.ds(row, _CH), :, :], xb.at[slot],
            xs.at[slot])
        cn = pltpu.make_async_copy(
            n_ref.at[pl.ds(b0, 2), pl.ds(row, _CH), :, :], nb.at[slot],
            ns.at[slot])
        cx.start()
        cn.start()

    in_copy(0, 0)
    for k in range(_NCHUNK):
        slot = k % 2
        if k + 1 < _NCHUNK:
            in_copy(k + 1, 1 - slot)
        pltpu.make_async_copy(xb.at[slot], xb.at[slot], xs.at[slot]).wait()
        pltpu.make_async_copy(nb.at[slot], nb.at[slot], ns.at[slot]).wait()
        if k >= 2:
            pltpu.make_async_copy(ob.at[slot], ob.at[slot], os_.at[slot]).wait()
        x = xb[slot]
        nz = nb[slot]
        c_lo = x > a0
        v0 = jnp.where(c_lo & (x <= rt), d, 0.0)
        v1 = jnp.where(c_lo & (x <= a59), d, 0.0) + jnp.where(
            (x > rt) & (x <= a0), d, 0.0)
        ob[slot] = x + v1 - (v0 + v1) * nz
        pltpu.make_async_copy(
            ob.at[slot], o_ref.at[pl.ds(b0, 2), pl.ds(k * _CH, _CH), :, :],
            os_.at[slot]).start()
    for slot in range(2):
        pltpu.make_async_copy(ob.at[slot], ob.at[slot], os_.at[slot]).wait()


def kernel(x, noise, w):
    xt = jnp.transpose(x, (0, 2, 3, 1))
    nt = jnp.transpose(noise, (0, 2, 3, 1))
    b, h, wd, c = xt.shape
    chunk = (2, _CH, wd, c)
    out = pl.pallas_call(
        _aq_kernel,
        grid=(b // 2,),
        in_specs=[
            pl.BlockSpec(memory_space=pltpu.SMEM),
            pl.BlockSpec(memory_space=pl.ANY),
            pl.BlockSpec(memory_space=pl.ANY),
        ],
        out_specs=pl.BlockSpec(memory_space=pl.ANY),
        out_shape=jax.ShapeDtypeStruct(xt.shape, x.dtype),
        scratch_shapes=[
            pltpu.VMEM((2,) + chunk, x.dtype),
            pltpu.VMEM((2,) + chunk, x.dtype),
            pltpu.VMEM((2,) + chunk, x.dtype),
            pltpu.SemaphoreType.DMA((2,)),
            pltpu.SemaphoreType.DMA((2,)),
            pltpu.SemaphoreType.DMA((2,)),
        ],
        compiler_params=pltpu.CompilerParams(
            dimension_semantics=("parallel",)),
    )(w, xt, nt)
    return jnp.transpose(out, (0, 3, 1, 2))
